# baseline (device time: 181795 ns/iter reference)
import jax
import jax.numpy as jnp
from jax import lax
from jax.experimental import pallas as pl
from jax.experimental.pallas import tpu as pltpu

DEPTH = 4


def kernel(Q, K, V):
    n_b, n_q, n_h, d = Q.shape
    k_per = K.shape[1]
    hd = n_h * d
    scale = d ** -0.5

    def body(q_ref, k_ref, v_ref, out_ref,
             kbuf, vbuf, ksems, vsems,
             accn_ref, accs_ref, rcvn_ref, rcvs_ref,
             send_sems, recv_sems):
        my_x = lax.axis_index("x")
        my_y = lax.axis_index("y")
        peer = (my_x, 1 - my_y)

        def kcopy(b, slot):
            return pltpu.make_async_copy(
                k_ref.at[b], kbuf.at[slot], ksems.at[slot])

        def vcopy(b, slot):
            return pltpu.make_async_copy(
                v_ref.at[b], vbuf.at[slot], vsems.at[slot])

        for i in range(DEPTH):
            kcopy(i, i).start()
            vcopy(i, i).start()

        eye = (lax.broadcasted_iota(jnp.int32, (n_h, n_h), 0)
               == lax.broadcasted_iota(jnp.int32, (n_h, n_h), 1))
        eye_bf = eye.astype(jnp.bfloat16)
        eye_f32 = eye.astype(jnp.float32)

        for b in range(n_b):
            slot = b % DEPTH
            kcopy(b, slot).wait()
            vcopy(b, slot).wait()

            q = q_ref[b, 0].astype(jnp.bfloat16)
            k2 = kbuf[slot].astype(jnp.bfloat16)
            v2 = vbuf[slot].astype(jnp.bfloat16)

            wq = (q[:, :, None] * eye_bf[:, None, :]).reshape(hd, n_h)

            s = lax.dot_general(
                k2, wq,
                dimension_numbers=(((1,), (0,)), ((), ())),
                preferred_element_type=jnp.float32,
            ) * scale
            m = jnp.max(s, axis=0, keepdims=True)
            p = jnp.exp(s - m)
            lsum = jnp.sum(p, axis=0, keepdims=True)

            c = lax.dot_general(
                p.astype(jnp.bfloat16), v2,
                dimension_numbers=(((0,), (0,)), ((), ())),
                preferred_element_type=jnp.float32,
            ).reshape(n_h, n_h, d)
            n = jnp.sum(c * eye_f32[:, :, None], axis=1)

            accn_ref[b] = n
            accs_ref[0, b] = m[0]
            accs_ref[1, b] = lsum[0]

            nxt = b + DEPTH
            if nxt < n_b:
                kcopy(nxt, slot).start()
                vcopy(nxt, slot).start()

        barrier = pltpu.get_barrier_semaphore()
        pl.semaphore_signal(barrier, inc=1, device_id=peer,
                            device_id_type=pl.DeviceIdType.MESH)
        pl.semaphore_wait(barrier, 1)

        rn = pltpu.make_async_remote_copy(
            src_ref=accn_ref, dst_ref=rcvn_ref,
            send_sem=send_sems.at[0], recv_sem=recv_sems.at[0],
            device_id=peer, device_id_type=pl.DeviceIdType.MESH)
        rs = pltpu.make_async_remote_copy(
            src_ref=accs_ref, dst_ref=rcvs_ref,
            send_sem=send_sems.at[1], recv_sem=recv_sems.at[1],
            device_id=peer, device_id_type=pl.DeviceIdType.MESH)
        rn.start()
        rs.start()
        rn.wait()
        rs.wait()

        m_loc = accs_ref[0]
        l_loc = accs_ref[1]
        m_rem = rcvs_ref[0]
        l_rem = rcvs_ref[1]
        m_new = jnp.maximum(m_loc, m_rem)
        a_loc = jnp.exp(m_loc - m_new)
        a_rem = jnp.exp(m_rem - m_new)
        l_new = a_loc * l_loc + a_rem * l_rem
        n_new = (a_loc[..., None] * accn_ref[...]
                 + a_rem[..., None] * rcvn_ref[...])
        out_ref[:, 0, :, :] = n_new / l_new[..., None]

    return pl.pallas_call(
        body,
        in_specs=[
            pl.BlockSpec(memory_space=pltpu.VMEM),
            pl.BlockSpec(memory_space=pl.ANY),
            pl.BlockSpec(memory_space=pl.ANY),
        ],
        out_specs=pl.BlockSpec(memory_space=pltpu.VMEM),
        out_shape=jax.ShapeDtypeStruct((n_b, n_q, n_h, d), jnp.float32),
        scratch_shapes=[
            pltpu.VMEM((DEPTH, k_per, hd), jnp.float32),
            pltpu.VMEM((DEPTH, k_per, hd), jnp.float32),
            pltpu.SemaphoreType.DMA((DEPTH,)),
            pltpu.SemaphoreType.DMA((DEPTH,)),
            pltpu.VMEM((n_b, n_h, d), jnp.float32),
            pltpu.VMEM((2, n_b, n_h), jnp.float32),
            pltpu.VMEM((n_b, n_h, d), jnp.float32),
            pltpu.VMEM((2, n_b, n_h), jnp.float32),
            pltpu.SemaphoreType.DMA((2,)),
            pltpu.SemaphoreType.DMA((2,)),
        ],
        compiler_params=pltpu.CompilerParams(
            collective_id=0,
            vmem_limit_bytes=100 * 1024 * 1024,
        ),
    )(Q, K.reshape(n_b, k_per, hd), V.reshape(n_b, k_per, hd))


# device time: 170116 ns/iter; 1.0687x vs baseline; 1.0687x over previous
import jax
import jax.numpy as jnp
from jax import lax
from jax.experimental import pallas as pl
from jax.experimental.pallas import tpu as pltpu

NSTREAM = 8
CHUNK = 512


def kernel(Q, K, V):
    n_b, n_q, n_h, d = Q.shape
    k_per = K.shape[1]
    hd = n_h * d
    per_b = k_per // CHUNK
    total = n_b * per_b

    def body(q_ref, k_ref, v_ref, out_ref, kbuf, vbuf, ksems, vsems, acc_ref):
        def kcopy(t, s):
            b, c = t // per_b, t % per_b
            return pltpu.make_async_copy(
                k_ref.at[b, pl.ds(c * CHUNK, CHUNK)], kbuf.at[s], ksems.at[s])

        def vcopy(t, s):
            b, c = t // per_b, t % per_b
            return pltpu.make_async_copy(
                v_ref.at[b, pl.ds(c * CHUNK, CHUNK)], vbuf.at[s], vsems.at[s])

        acc_ref[...] = jnp.zeros_like(acc_ref)
        for t in range(NSTREAM):
            kcopy(t, t).start()
            vcopy(t, t).start()
        for t in range(total):
            s = t % NSTREAM
            kcopy(t, s).wait()
            vcopy(t, s).wait()
            acc_ref[...] += (jnp.sum(kbuf[s], axis=0, keepdims=True)
                             + jnp.sum(vbuf[s], axis=0, keepdims=True))
            nxt = t + NSTREAM
            if nxt < total:
                kcopy(nxt, s).start()
                vcopy(nxt, s).start()
        out_ref[0:1] = acc_ref[...].reshape(1, 1, n_h, d) + q_ref[0:1]

    return pl.pallas_call(
        body,
        in_specs=[
            pl.BlockSpec(memory_space=pltpu.VMEM),
            pl.BlockSpec(memory_space=pl.ANY),
            pl.BlockSpec(memory_space=pl.ANY),
        ],
        out_specs=pl.BlockSpec(memory_space=pltpu.VMEM),
        out_shape=jax.ShapeDtypeStruct((n_b, n_q, n_h, d), jnp.float32),
        scratch_shapes=[
            pltpu.VMEM((NSTREAM, CHUNK, hd), jnp.float32),
            pltpu.VMEM((NSTREAM, CHUNK, hd), jnp.float32),
            pltpu.SemaphoreType.DMA((NSTREAM,)),
            pltpu.SemaphoreType.DMA((NSTREAM,)),
            pltpu.VMEM((1, hd), jnp.float32),
        ],
        compiler_params=pltpu.CompilerParams(
            vmem_limit_bytes=100 * 1024 * 1024,
        ),
    )(Q, K.reshape(n_b, k_per, hd), V.reshape(n_b, k_per, hd))


# device time: 50227 ns/iter; 3.6195x vs baseline; 3.3869x over previous
import jax
import jax.numpy as jnp
from jax import lax
from jax.experimental import pallas as pl
from jax.experimental.pallas import tpu as pltpu


def kernel(Q, K, V):
    n_b, n_q, n_h, d = Q.shape
    k_per = K.shape[1]
    hd = n_h * d
    scale = d ** -0.5

    def body(q_ref, k_ref, v_ref, out_ref,
             accn_ref, accs_ref, rcvn_ref, rcvs_ref,
             send_sems, recv_sems):
        b = pl.program_id(0)
        my_x = lax.axis_index("x")
        my_y = lax.axis_index("y")
        peer = (my_x, 1 - my_y)

        q = q_ref[0, 0].astype(jnp.bfloat16)
        k2 = k_ref[0].reshape(hd, k_per).astype(jnp.bfloat16)
        v2 = v_ref[0].reshape(hd, k_per).astype(jnp.bfloat16)

        eye = (lax.broadcasted_iota(jnp.int32, (n_h, n_h), 0)
               == lax.broadcasted_iota(jnp.int32, (n_h, n_h), 1))

        wq = (eye.astype(jnp.bfloat16)[:, :, None]
              * q[None, :, :]).reshape(n_h, hd)

        s = lax.dot_general(
            wq, k2,
            dimension_numbers=(((1,), (0,)), ((), ())),
            preferred_element_type=jnp.float32,
        ) * scale
        m = jnp.max(s, axis=1, keepdims=True)
        p = jnp.exp(s - m)
        lsum = jnp.sum(p, axis=1, keepdims=True)

        c = lax.dot_general(
            v2, p.astype(jnp.bfloat16),
            dimension_numbers=(((1,), (1,)), ((), ())),
            preferred_element_type=jnp.float32,
        ).reshape(n_h, d, n_h)
        n = jnp.sum(c * eye.astype(jnp.float32)[:, None, :], axis=2)

        accn_ref[b] = n
        accs_ref[0, b] = m[:, 0]
        accs_ref[1, b] = lsum[:, 0]

        @pl.when(b == n_b - 1)
        def _():
            barrier = pltpu.get_barrier_semaphore()
            pl.semaphore_signal(barrier, inc=1, device_id=peer,
                                device_id_type=pl.DeviceIdType.MESH)
            pl.semaphore_wait(barrier, 1)

            rn = pltpu.make_async_remote_copy(
                src_ref=accn_ref, dst_ref=rcvn_ref,
                send_sem=send_sems.at[0], recv_sem=recv_sems.at[0],
                device_id=peer, device_id_type=pl.DeviceIdType.MESH)
            rs = pltpu.make_async_remote_copy(
                src_ref=accs_ref, dst_ref=rcvs_ref,
                send_sem=send_sems.at[1], recv_sem=recv_sems.at[1],
                device_id=peer, device_id_type=pl.DeviceIdType.MESH)
            rn.start()
            rs.start()
            rn.wait()
            rs.wait()

            m_loc = accs_ref[0]
            l_loc = accs_ref[1]
            m_rem = rcvs_ref[0]
            l_rem = rcvs_ref[1]
            m_new = jnp.maximum(m_loc, m_rem)
            a_loc = jnp.exp(m_loc - m_new)
            a_rem = jnp.exp(m_rem - m_new)
            l_new = a_loc * l_loc + a_rem * l_rem
            n_new = (a_loc[..., None] * accn_ref[...]
                     + a_rem[..., None] * rcvn_ref[...])
            out_ref[:, 0, :, :] = n_new / l_new[..., None]

    return pl.pallas_call(
        body,
        grid=(n_b,),
        in_specs=[
            pl.BlockSpec((1, 1, n_h, d), lambda b: (b, 0, 0, 0)),
            pl.BlockSpec((1, n_h, d, k_per), lambda b: (b, 0, 0, 0)),
            pl.BlockSpec((1, n_h, d, k_per), lambda b: (b, 0, 0, 0)),
        ],
        out_specs=pl.BlockSpec((n_b, 1, n_h, d), lambda b: (0, 0, 0, 0)),
        out_shape=jax.ShapeDtypeStruct((n_b, n_q, n_h, d), jnp.float32),
        scratch_shapes=[
            pltpu.VMEM((n_b, n_h, d), jnp.float32),
            pltpu.VMEM((2, n_b, n_h), jnp.float32),
            pltpu.VMEM((n_b, n_h, d), jnp.float32),
            pltpu.VMEM((2, n_b, n_h), jnp.float32),
            pltpu.SemaphoreType.DMA((2,)),
            pltpu.SemaphoreType.DMA((2,)),
        ],
        compiler_params=pltpu.CompilerParams(
            collective_id=0,
            dimension_semantics=("arbitrary",),
            vmem_limit_bytes=100 * 1024 * 1024,
        ),
    )(Q, jnp.transpose(K, (0, 2, 3, 1)), jnp.transpose(V, (0, 2, 3, 1)))


# device time: 36129 ns/iter; 5.0318x vs baseline; 1.3902x over previous
import jax
import jax.numpy as jnp
from jax import lax
from jax.experimental import pallas as pl
from jax.experimental.pallas import tpu as pltpu

N_X = 2
N_Y = 2


def kernel(Q, K, V):
    n_b, n_q, n_h, d = Q.shape
    k_per = K.shape[1]
    hd = n_h * d
    scale = d ** -0.5
    half = n_b // N_X

    def body(q_ref, k_ref, v_ref, out_ref,
             kbuf, vbuf, ksems, vsems,
             accn_ref, accs_ref, rcvn_ref, rcvs_ref,
             send_sems, recv_sems):
        my_x = lax.axis_index("x")
        my_y = lax.axis_index("y")
        xpeer = (1 - my_x, my_y)
        ypeer = (my_x, 1 - my_y)
        b0 = my_x * half

        def kcopy(i, slot):
            return pltpu.make_async_copy(
                k_ref.at[b0 + i], kbuf.at[slot], ksems.at[slot])

        def vcopy(i, slot):
            return pltpu.make_async_copy(
                v_ref.at[b0 + i], vbuf.at[slot], vsems.at[slot])

        kcopy(0, 0).start()
        vcopy(0, 0).start()
        kcopy(1, 1).start()
        vcopy(1, 1).start()

        eye = (lax.broadcasted_iota(jnp.int32, (n_h, n_h), 0)
               == lax.broadcasted_iota(jnp.int32, (n_h, n_h), 1))
        eye_bf = eye.astype(jnp.bfloat16)
        eye_f32 = eye.astype(jnp.float32)

        for i in range(half):
            slot = i % 2
            kcopy(i, slot).wait()
            vcopy(i, slot).wait()

            q = q_ref[b0 + i, 0].astype(jnp.bfloat16)
            k2 = kbuf[slot].reshape(hd, k_per).astype(jnp.bfloat16)
            v2 = vbuf[slot].reshape(hd, k_per).astype(jnp.bfloat16)

            wq = (eye_bf[:, :, None] * q[None, :, :]).reshape(n_h, hd)

            s = lax.dot_general(
                wq, k2,
                dimension_numbers=(((1,), (0,)), ((), ())),
                preferred_element_type=jnp.float32,
            ) * scale
            m = jnp.max(s, axis=1, keepdims=True)
            p = jnp.exp(s - m)
            lsum = jnp.sum(p, axis=1, keepdims=True)

            c = lax.dot_general(
                v2, p.astype(jnp.bfloat16),
                dimension_numbers=(((1,), (1,)), ((), ())),
                preferred_element_type=jnp.float32,
            ).reshape(n_h, d, n_h)
            n = jnp.sum(c * eye_f32[:, None, :], axis=2)

            accn_ref[b0 + i] = n
            accs_ref[0, b0 + i] = m[:, 0]
            accs_ref[1, b0 + i] = lsum[:, 0]

            nxt = i + 2
            if nxt < half:
                kcopy(nxt, slot).start()
                vcopy(nxt, slot).start()

        barrier = pltpu.get_barrier_semaphore()
        for nbr in (xpeer, ypeer):
            pl.semaphore_signal(barrier, inc=1, device_id=nbr,
                                device_id_type=pl.DeviceIdType.MESH)
        pl.semaphore_wait(barrier, 2)

        xn = pltpu.make_async_remote_copy(
            src_ref=accn_ref.at[pl.ds(b0, half)],
            dst_ref=accn_ref.at[pl.ds(b0, half)],
            send_sem=send_sems.at[0], recv_sem=recv_sems.at[0],
            device_id=xpeer, device_id_type=pl.DeviceIdType.MESH)
        xs = pltpu.make_async_remote_copy(
            src_ref=accs_ref.at[:, pl.ds(b0, half)],
            dst_ref=accs_ref.at[:, pl.ds(b0, half)],
            send_sem=send_sems.at[1], recv_sem=recv_sems.at[1],
            device_id=xpeer, device_id_type=pl.DeviceIdType.MESH)
        xn.start()
        xs.start()
        xn.wait()
        xs.wait()

        yn = pltpu.make_async_remote_copy(
            src_ref=accn_ref, dst_ref=rcvn_ref,
            send_sem=send_sems.at[2], recv_sem=recv_sems.at[2],
            device_id=ypeer, device_id_type=pl.DeviceIdType.MESH)
        ys = pltpu.make_async_remote_copy(
            src_ref=accs_ref, dst_ref=rcvs_ref,
            send_sem=send_sems.at[3], recv_sem=recv_sems.at[3],
            device_id=ypeer, device_id_type=pl.DeviceIdType.MESH)
        yn.start()
        ys.start()
        yn.wait()
        ys.wait()

        m_loc = accs_ref[0]
        l_loc = accs_ref[1]
        m_rem = rcvs_ref[0]
        l_rem = rcvs_ref[1]
        m_new = jnp.maximum(m_loc, m_rem)
        a_loc = jnp.exp(m_loc - m_new)
        a_rem = jnp.exp(m_rem - m_new)
        l_new = a_loc * l_loc + a_rem * l_rem
        n_new = (a_loc[..., None] * accn_ref[...]
                 + a_rem[..., None] * rcvn_ref[...])
        out_ref[:, 0, :, :] = n_new / l_new[..., None]

    return pl.pallas_call(
        body,
        in_specs=[
            pl.BlockSpec(memory_space=pltpu.VMEM),
            pl.BlockSpec(memory_space=pl.ANY),
            pl.BlockSpec(memory_space=pl.ANY),
        ],
        out_specs=pl.BlockSpec(memory_space=pltpu.VMEM),
        out_shape=jax.ShapeDtypeStruct((n_b, n_q, n_h, d), jnp.float32),
        scratch_shapes=[
            pltpu.VMEM((2, n_h, d, k_per), jnp.float32),
            pltpu.VMEM((2, n_h, d, k_per), jnp.float32),
            pltpu.SemaphoreType.DMA((2,)),
            pltpu.SemaphoreType.DMA((2,)),
            pltpu.VMEM((n_b, n_h, d), jnp.float32),
            pltpu.VMEM((2, n_b, n_h), jnp.float32),
            pltpu.VMEM((n_b, n_h, d), jnp.float32),
            pltpu.VMEM((2, n_b, n_h), jnp.float32),
            pltpu.SemaphoreType.DMA((4,)),
            pltpu.SemaphoreType.DMA((4,)),
        ],
        compiler_params=pltpu.CompilerParams(
            collective_id=0,
            vmem_limit_bytes=100 * 1024 * 1024,
        ),
    )(Q, jnp.transpose(K, (0, 2, 3, 1)), jnp.transpose(V, (0, 2, 3, 1)))


# device time: 36066 ns/iter; 5.0406x vs baseline; 1.0017x over previous
import jax
import jax.numpy as jnp
from jax import lax
from jax.experimental import pallas as pl
from jax.experimental.pallas import tpu as pltpu

N_X = 2
N_Y = 2
DEPTH = 4


def kernel(Q, K, V):
    n_b, n_q, n_h, d = Q.shape
    k_per = K.shape[1]
    hd = n_h * d
    scale = d ** -0.5
    half = n_b // N_X

    def body(q_ref, k_ref, v_ref, out_ref,
             kbuf, vbuf, ksems, vsems,
             accn_ref, accs_ref, rcvn_ref, rcvs_ref,
             send_sems, recv_sems):
        my_x = lax.axis_index("x")
        my_y = lax.axis_index("y")
        xpeer = (1 - my_x, my_y)
        ypeer = (my_x, 1 - my_y)
        b0 = my_x * half

        def kcopy(i, slot):
            return pltpu.make_async_copy(
                k_ref.at[b0 + i], kbuf.at[slot], ksems.at[slot])

        def vcopy(i, slot):
            return pltpu.make_async_copy(
                v_ref.at[b0 + i], vbuf.at[slot], vsems.at[slot])

        barrier = pltpu.get_barrier_semaphore()
        for nbr in (xpeer, ypeer):
            pl.semaphore_signal(barrier, inc=1, device_id=nbr,
                                device_id_type=pl.DeviceIdType.MESH)
        pl.semaphore_wait(barrier, 2)

        for s in range(DEPTH):
            kcopy(s, s).start()
            vcopy(s, s).start()

        eye = (lax.broadcasted_iota(jnp.int32, (n_h, n_h), 0)
               == lax.broadcasted_iota(jnp.int32, (n_h, n_h), 1))
        eye_bf = eye.astype(jnp.bfloat16)
        eye_f32 = eye.astype(jnp.float32)

        for i in range(half):
            slot = i % DEPTH
            kcopy(i, slot).wait()
            vcopy(i, slot).wait()

            q = q_ref[b0 + i, 0].astype(jnp.bfloat16)
            k2 = kbuf[slot].reshape(hd, k_per).astype(jnp.bfloat16)
            v2 = vbuf[slot].reshape(hd, k_per).astype(jnp.bfloat16)

            wq = (eye_bf[:, :, None] * q[None, :, :]).reshape(n_h, hd)

            s = lax.dot_general(
                wq, k2,
                dimension_numbers=(((1,), (0,)), ((), ())),
                preferred_element_type=jnp.float32,
            ) * scale
            m = jnp.max(s, axis=1, keepdims=True)
            p = jnp.exp(s - m)
            lsum = jnp.sum(p, axis=1, keepdims=True)

            c = lax.dot_general(
                v2, p.astype(jnp.bfloat16),
                dimension_numbers=(((1,), (1,)), ((), ())),
                preferred_element_type=jnp.float32,
            ).reshape(n_h, d, n_h)
            n = jnp.sum(c * eye_f32[:, None, :], axis=2)

            accn_ref[b0 + i] = n
            accs_ref[0, b0 + i] = m[:, 0]
            accs_ref[1, b0 + i] = lsum[:, 0]

            nxt = i + DEPTH
            if nxt < half:
                kcopy(nxt, slot).start()
                vcopy(nxt, slot).start()

        xn = pltpu.make_async_remote_copy(
            src_ref=accn_ref.at[pl.ds(b0, half)],
            dst_ref=accn_ref.at[pl.ds(b0, half)],
            send_sem=send_sems.at[0], recv_sem=recv_sems.at[0],
            device_id=xpeer, device_id_type=pl.DeviceIdType.MESH)
        xs = pltpu.make_async_remote_copy(
            src_ref=accs_ref.at[:, pl.ds(b0, half)],
            dst_ref=accs_ref.at[:, pl.ds(b0, half)],
            send_sem=send_sems.at[1], recv_sem=recv_sems.at[1],
            device_id=xpeer, device_id_type=pl.DeviceIdType.MESH)
        xn.start()
        xs.start()
        xn.wait()
        xs.wait()

        yn = pltpu.make_async_remote_copy(
            src_ref=accn_ref, dst_ref=rcvn_ref,
            send_sem=send_sems.at[2], recv_sem=recv_sems.at[2],
            device_id=ypeer, device_id_type=pl.DeviceIdType.MESH)
        ys = pltpu.make_async_remote_copy(
            src_ref=accs_ref, dst_ref=rcvs_ref,
            send_sem=send_sems.at[3], recv_sem=recv_sems.at[3],
            device_id=ypeer, device_id_type=pl.DeviceIdType.MESH)
        yn.start()
        ys.start()
        yn.wait()
        ys.wait()

        m_loc = accs_ref[0]
        l_loc = accs_ref[1]
        m_rem = rcvs_ref[0]
        l_rem = rcvs_ref[1]
        m_new = jnp.maximum(m_loc, m_rem)
        a_loc = jnp.exp(m_loc - m_new)
        a_rem = jnp.exp(m_rem - m_new)
        l_new = a_loc * l_loc + a_rem * l_rem
        n_new = (a_loc[..., None] * accn_ref[...]
                 + a_rem[..., None] * rcvn_ref[...])
        out_ref[:, 0, :, :] = n_new / l_new[..., None]

    return pl.pallas_call(
        body,
        in_specs=[
            pl.BlockSpec(memory_space=pltpu.VMEM),
            pl.BlockSpec(memory_space=pl.ANY),
            pl.BlockSpec(memory_space=pl.ANY),
        ],
        out_specs=pl.BlockSpec(memory_space=pltpu.VMEM),
        out_shape=jax.ShapeDtypeStruct((n_b, n_q, n_h, d), jnp.float32),
        scratch_shapes=[
            pltpu.VMEM((DEPTH, n_h, d, k_per), jnp.float32),
            pltpu.VMEM((DEPTH, n_h, d, k_per), jnp.float32),
            pltpu.SemaphoreType.DMA((DEPTH,)),
            pltpu.SemaphoreType.DMA((DEPTH,)),
            pltpu.VMEM((n_b, n_h, d), jnp.float32),
            pltpu.VMEM((2, n_b, n_h), jnp.float32),
            pltpu.VMEM((n_b, n_h, d), jnp.float32),
            pltpu.VMEM((2, n_b, n_h), jnp.float32),
            pltpu.SemaphoreType.DMA((4,)),
            pltpu.SemaphoreType.DMA((4,)),
        ],
        compiler_params=pltpu.CompilerParams(
            collective_id=0,
            vmem_limit_bytes=100 * 1024 * 1024,
        ),
    )(Q, jnp.transpose(K, (0, 2, 3, 1)), jnp.transpose(V, (0, 2, 3, 1)))


# device time: 34670 ns/iter; 5.2436x vs baseline; 1.0403x over previous
import jax
import jax.numpy as jnp
from jax import lax
from jax.experimental import pallas as pl
from jax.experimental.pallas import tpu as pltpu

N_X = 2
N_Y = 2
DEPTH = 4


def kernel(Q, K, V):
    n_b, n_q, n_h, d = Q.shape
    k_per = K.shape[1]
    hd = n_h * d
    scale = d ** -0.5
    half = n_b // N_X

    def body(q_ref, k_ref, v_ref, out_ref,
             kbuf, vbuf, ksems, vsems,
             accn_ref, accs_ref, rcvn_ref, rcvs_ref,
             send_sems, recv_sems):
        my_x = lax.axis_index("x")
        my_y = lax.axis_index("y")
        xpeer = (1 - my_x, my_y)
        ypeer = (my_x, 1 - my_y)
        dpeer = (1 - my_x, 1 - my_y)
        b0 = my_x * half

        def kcopy(i, slot):
            return pltpu.make_async_copy(
                k_ref.at[b0 + i], kbuf.at[slot], ksems.at[slot])

        def vcopy(i, slot):
            return pltpu.make_async_copy(
                v_ref.at[b0 + i], vbuf.at[slot], vsems.at[slot])

        barrier = pltpu.get_barrier_semaphore()
        for nbr in (xpeer, ypeer, dpeer):
            pl.semaphore_signal(barrier, inc=1, device_id=nbr,
                                device_id_type=pl.DeviceIdType.MESH)
        pl.semaphore_wait(barrier, 3)

        for s in range(DEPTH):
            kcopy(s, s).start()
            vcopy(s, s).start()

        eye = (lax.broadcasted_iota(jnp.int32, (n_h, n_h), 0)
               == lax.broadcasted_iota(jnp.int32, (n_h, n_h), 1))
        eye_bf = eye.astype(jnp.bfloat16)
        eye_f32 = eye.astype(jnp.float32)

        for i in range(half):
            slot = i % DEPTH
            kcopy(i, slot).wait()
            vcopy(i, slot).wait()

            q = q_ref[b0 + i, 0].astype(jnp.bfloat16)
            k2 = kbuf[slot].reshape(hd, k_per).astype(jnp.bfloat16)
            v2 = vbuf[slot].reshape(hd, k_per).astype(jnp.bfloat16)

            wq = (eye_bf[:, :, None] * q[None, :, :]).reshape(n_h, hd)

            s = lax.dot_general(
                wq, k2,
                dimension_numbers=(((1,), (0,)), ((), ())),
                preferred_element_type=jnp.float32,
            ) * scale
            m = jnp.max(s, axis=1, keepdims=True)
            p = jnp.exp(s - m)
            lsum = jnp.sum(p, axis=1, keepdims=True)

            c = lax.dot_general(
                v2, p.astype(jnp.bfloat16),
                dimension_numbers=(((1,), (1,)), ((), ())),
                preferred_element_type=jnp.float32,
            ).reshape(n_h, d, n_h)
            n = jnp.sum(c * eye_f32[:, None, :], axis=2)

            accn_ref[b0 + i] = n
            accs_ref[0, b0 + i] = m[:, 0]
            accs_ref[1, b0 + i] = lsum[:, 0]

            nxt = i + DEPTH
            if nxt < half:
                kcopy(nxt, slot).start()
                vcopy(nxt, slot).start()

        rdmas = []
        for idx, (peer, nbuf, sbuf) in enumerate((
                (xpeer, accn_ref, accs_ref),
                (ypeer, rcvn_ref, rcvs_ref),
                (dpeer, rcvn_ref, rcvs_ref))):
            rdmas.append(pltpu.make_async_remote_copy(
                src_ref=accn_ref.at[pl.ds(b0, half)],
                dst_ref=nbuf.at[pl.ds(b0, half)],
                send_sem=send_sems.at[2 * idx], recv_sem=recv_sems.at[2 * idx],
                device_id=peer, device_id_type=pl.DeviceIdType.MESH))
            rdmas.append(pltpu.make_async_remote_copy(
                src_ref=accs_ref.at[:, pl.ds(b0, half)],
                dst_ref=sbuf.at[:, pl.ds(b0, half)],
                send_sem=send_sems.at[2 * idx + 1],
                recv_sem=recv_sems.at[2 * idx + 1],
                device_id=peer, device_id_type=pl.DeviceIdType.MESH))
        for r in rdmas:
            r.start()
        for r in rdmas:
            r.wait()

        m_loc = accs_ref[0]
        l_loc = accs_ref[1]
        m_rem = rcvs_ref[0]
        l_rem = rcvs_ref[1]
        m_new = jnp.maximum(m_loc, m_rem)
        a_loc = jnp.exp(m_loc - m_new)
        a_rem = jnp.exp(m_rem - m_new)
        l_new = a_loc * l_loc + a_rem * l_rem
        n_new = (a_loc[..., None] * accn_ref[...]
                 + a_rem[..., None] * rcvn_ref[...])
        out_ref[:, 0, :, :] = n_new / l_new[..., None]

    return pl.pallas_call(
        body,
        in_specs=[
            pl.BlockSpec(memory_space=pltpu.VMEM),
            pl.BlockSpec(memory_space=pl.ANY),
            pl.BlockSpec(memory_space=pl.ANY),
        ],
        out_specs=pl.BlockSpec(memory_space=pltpu.VMEM),
        out_shape=jax.ShapeDtypeStruct((n_b, n_q, n_h, d), jnp.float32),
        scratch_shapes=[
            pltpu.VMEM((DEPTH, n_h, d, k_per), jnp.float32),
            pltpu.VMEM((DEPTH, n_h, d, k_per), jnp.float32),
            pltpu.SemaphoreType.DMA((DEPTH,)),
            pltpu.SemaphoreType.DMA((DEPTH,)),
            pltpu.VMEM((n_b, n_h, d), jnp.float32),
            pltpu.VMEM((2, n_b, n_h), jnp.float32),
            pltpu.VMEM((n_b, n_h, d), jnp.float32),
            pltpu.VMEM((2, n_b, n_h), jnp.float32),
            pltpu.SemaphoreType.DMA((6,)),
            pltpu.SemaphoreType.DMA((6,)),
        ],
        compiler_params=pltpu.CompilerParams(
            collective_id=0,
            vmem_limit_bytes=100 * 1024 * 1024,
        ),
    )(Q, jnp.transpose(K, (0, 2, 3, 1)), jnp.transpose(V, (0, 2, 3, 1)))


# device time: 32946 ns/iter; 5.5180x vs baseline; 1.0523x over previous
import jax
import jax.numpy as jnp
from jax import lax
from jax.experimental import pallas as pl
from jax.experimental.pallas import tpu as pltpu

N_X = 2
N_Y = 2
DEPTH = 4


def kernel(Q, K, V):
    n_b, n_q, n_h, d = Q.shape
    k_per = K.shape[1]
    hd = n_h * d
    scale = d ** -0.5
    half = n_b // N_X

    def body(q_ref, k_ref, v_ref, out_ref,
             kbuf, vbuf, ksems, vsems,
             accn_ref, accs_ref, rcvn_ref, rcvs_ref,
             send_sems, recv_sems):
        my_x = lax.axis_index("x")
        my_y = lax.axis_index("y")
        xpeer = (1 - my_x, my_y)
        ypeer = (my_x, 1 - my_y)
        dpeer = (1 - my_x, 1 - my_y)
        b0 = my_x * half

        def kcopy(i, slot):
            return pltpu.make_async_copy(
                k_ref.at[b0 + i], kbuf.at[slot], ksems.at[slot])

        def vcopy(i, slot):
            return pltpu.make_async_copy(
                v_ref.at[b0 + i], vbuf.at[slot], vsems.at[slot])

        barrier = pltpu.get_barrier_semaphore()
        for nbr in (xpeer, ypeer, dpeer):
            pl.semaphore_signal(barrier, inc=1, device_id=nbr,
                                device_id_type=pl.DeviceIdType.MESH)
        pl.semaphore_wait(barrier, 3)

        for s in range(DEPTH):
            kcopy(s, s).start()
            vcopy(s, s).start()

        eye = (lax.broadcasted_iota(jnp.int32, (n_h, n_h), 0)
               == lax.broadcasted_iota(jnp.int32, (n_h, n_h), 1))
        eye_bf = eye.astype(jnp.bfloat16)
        eye_f32 = eye.astype(jnp.float32)

        for i in range(half):
            slot = i % DEPTH
            kcopy(i, slot).wait()
            vcopy(i, slot).wait()

            q3 = q_ref[b0 + i]
            k3 = kbuf[slot]
            v3 = vbuf[slot]

            s = lax.dot_general(
                jnp.transpose(q3, (1, 0, 2)), k3,
                dimension_numbers=(((2,), (1,)), ((0,), (0,))),
                preferred_element_type=jnp.float32,
                precision=lax.Precision.DEFAULT,
            ) * scale
            m = jnp.max(s, axis=2, keepdims=True)
            p = jnp.exp(s - m)
            lsum = jnp.sum(p, axis=2, keepdims=True)

            n = lax.dot_general(
                p, v3,
                dimension_numbers=(((2,), (2,)), ((0,), (0,))),
                preferred_element_type=jnp.float32,
                precision=lax.Precision.DEFAULT,
            )[:, 0, :]
            m = m[:, 0, :]
            lsum = lsum[:, 0, :]

            accn_ref[b0 + i] = n
            accs_ref[0, b0 + i] = m[:, 0]
            accs_ref[1, b0 + i] = lsum[:, 0]

            nxt = i + DEPTH
            if nxt < half:
                kcopy(nxt, slot).start()
                vcopy(nxt, slot).start()

        rdmas = []
        for idx, (peer, nbuf, sbuf) in enumerate((
                (xpeer, accn_ref, accs_ref),
                (ypeer, rcvn_ref, rcvs_ref),
                (dpeer, rcvn_ref, rcvs_ref))):
            rdmas.append(pltpu.make_async_remote_copy(
                src_ref=accn_ref.at[pl.ds(b0, half)],
                dst_ref=nbuf.at[pl.ds(b0, half)],
                send_sem=send_sems.at[2 * idx], recv_sem=recv_sems.at[2 * idx],
                device_id=peer, device_id_type=pl.DeviceIdType.MESH))
            rdmas.append(pltpu.make_async_remote_copy(
                src_ref=accs_ref.at[:, pl.ds(b0, half)],
                dst_ref=sbuf.at[:, pl.ds(b0, half)],
                send_sem=send_sems.at[2 * idx + 1],
                recv_sem=recv_sems.at[2 * idx + 1],
                device_id=peer, device_id_type=pl.DeviceIdType.MESH))
        for r in rdmas:
            r.start()
        for r in rdmas:
            r.wait()

        m_loc = accs_ref[0]
        l_loc = accs_ref[1]
        m_rem = rcvs_ref[0]
        l_rem = rcvs_ref[1]
        m_new = jnp.maximum(m_loc, m_rem)
        a_loc = jnp.exp(m_loc - m_new)
        a_rem = jnp.exp(m_rem - m_new)
        l_new = a_loc * l_loc + a_rem * l_rem
        n_new = (a_loc[..., None] * accn_ref[...]
                 + a_rem[..., None] * rcvn_ref[...])
        out_ref[:, 0, :, :] = n_new / l_new[..., None]

    return pl.pallas_call(
        body,
        in_specs=[
            pl.BlockSpec(memory_space=pltpu.VMEM),
            pl.BlockSpec(memory_space=pl.ANY),
            pl.BlockSpec(memory_space=pl.ANY),
        ],
        out_specs=pl.BlockSpec(memory_space=pltpu.VMEM),
        out_shape=jax.ShapeDtypeStruct((n_b, n_q, n_h, d), jnp.float32),
        scratch_shapes=[
            pltpu.VMEM((DEPTH, n_h, d, k_per), jnp.float32),
            pltpu.VMEM((DEPTH, n_h, d, k_per), jnp.float32),
            pltpu.SemaphoreType.DMA((DEPTH,)),
            pltpu.SemaphoreType.DMA((DEPTH,)),
            pltpu.VMEM((n_b, n_h, d), jnp.float32),
            pltpu.VMEM((2, n_b, n_h), jnp.float32),
            pltpu.VMEM((n_b, n_h, d), jnp.float32),
            pltpu.VMEM((2, n_b, n_h), jnp.float32),
            pltpu.SemaphoreType.DMA((6,)),
            pltpu.SemaphoreType.DMA((6,)),
        ],
        compiler_params=pltpu.CompilerParams(
            collective_id=0,
            vmem_limit_bytes=100 * 1024 * 1024,
        ),
    )(Q, jnp.transpose(K, (0, 2, 3, 1)), jnp.transpose(V, (0, 2, 3, 1)))
